# initial kernel scaffold (unmeasured)
import jax
import jax.numpy as jnp
from jax import lax
from jax.experimental import pallas as pl
from jax.experimental.pallas import tpu as pltpu

N_DEV = 32
H_LOC = 8
DH = 128
SQ = 512
SKV = 2048
D_LOC = H_LOC * DH
SCALE = 0.08838834764831843
LOG2_N = 5

_RS_SLOT = [0, 256, 384, 448, 480]
_RS_ROWS = 496


def kernel(x, Wq, Wo, K_ext, V_ext):
    def body(x_ref, wq_ref, wo_ref, k_hbm, v_hbm, out_ref,
             k_vmem, v_vmem, attn_ref, recv_ref,
             copy_sems, send_sems, recv_sems):
        my = lax.axis_index("i")
        h0 = my * H_LOC

        kcp = pltpu.make_async_copy(
            k_hbm.at[0, :, pl.ds(h0, H_LOC), :], k_vmem, copy_sems.at[0])
        vcp = pltpu.make_async_copy(
            v_hbm.at[0, :, pl.ds(h0, H_LOC), :], v_vmem, copy_sems.at[1])
        kcp.start()
        vcp.start()

        q = jnp.dot(x_ref[0].astype(jnp.bfloat16),
                    wq_ref[:].astype(jnp.bfloat16),
                    preferred_element_type=jnp.float32)

        kcp.wait()
        vcp.wait()

        for h in range(H_LOC):
            qh = q[:, h * DH:(h + 1) * DH].astype(jnp.bfloat16)
            kh = k_vmem[:, h, :].astype(jnp.bfloat16)
            s = lax.dot_general(
                qh, kh, (((1,), (1,)), ((), ())),
                preferred_element_type=jnp.float32) * SCALE
            m = jnp.max(s, axis=1, keepdims=True)
            p = jnp.exp(s - m)
            l = jnp.sum(p, axis=1, keepdims=True)
            vh = v_vmem[:, h, :].astype(jnp.bfloat16)
            o = jnp.dot(p.astype(jnp.bfloat16), vh,
                        preferred_element_type=jnp.float32)
            attn_ref[:, h * DH:(h + 1) * DH] = (o / l).astype(jnp.bfloat16)

        out_ref[0] = jnp.dot(attn_ref[:], wo_ref[:].astype(jnp.bfloat16),
                             preferred_element_type=jnp.float32)

        off = 0
        ln = SQ
        for k in range(LOG2_N):
            half = ln // 2
            b = jnp.bitwise_and(jnp.right_shift(my, k), 1)
            partner = jnp.bitwise_xor(my, 1 << k)
            send_off = off + (1 - b) * half
            keep_off = off + b * half
            rdma = pltpu.make_async_remote_copy(
                src_ref=out_ref.at[0, pl.ds(send_off, half), :],
                dst_ref=recv_ref.at[pl.ds(_RS_SLOT[k], half), :],
                send_sem=send_sems.at[k],
                recv_sem=recv_sems.at[k],
                device_id=(partner,),
                device_id_type=pl.DeviceIdType.MESH,
            )
            rdma.start()
            rdma.wait()
            out_ref[0, pl.ds(keep_off, half), :] = (
                out_ref[0, pl.ds(keep_off, half), :]
                + recv_ref[_RS_SLOT[k]:_RS_SLOT[k] + half, :])
            off = keep_off
            ln = half

        for i, k in enumerate(range(LOG2_N - 1, -1, -1)):
            b = jnp.bitwise_and(jnp.right_shift(my, k), 1)
            partner = jnp.bitwise_xor(my, 1 << k)
            rdma = pltpu.make_async_remote_copy(
                src_ref=out_ref.at[0, pl.ds(off, ln), :],
                dst_ref=out_ref.at[0, pl.ds(off, ln), :],
                send_sem=send_sems.at[LOG2_N + i],
                recv_sem=recv_sems.at[LOG2_N + i],
                device_id=(partner,),
                device_id_type=pl.DeviceIdType.MESH,
            )
            rdma.start()
            rdma.wait()
            off = off - b * ln
            ln = ln * 2

    return pl.pallas_call(
        body,
        out_shape=jax.ShapeDtypeStruct((1, SQ, D_LOC), jnp.float32),
        in_specs=[
            pl.BlockSpec(memory_space=pltpu.VMEM),
            pl.BlockSpec(memory_space=pltpu.VMEM),
            pl.BlockSpec(memory_space=pltpu.VMEM),
            pl.BlockSpec(memory_space=pltpu.ANY),
            pl.BlockSpec(memory_space=pltpu.ANY),
        ],
        out_specs=pl.BlockSpec(memory_space=pltpu.VMEM),
        scratch_shapes=[
            pltpu.VMEM((SKV, H_LOC, DH), jnp.float32),
            pltpu.VMEM((SKV, H_LOC, DH), jnp.float32),
            pltpu.VMEM((SQ, D_LOC), jnp.bfloat16),
            pltpu.VMEM((_RS_ROWS, D_LOC), jnp.float32),
            pltpu.SemaphoreType.DMA((2,)),
            pltpu.SemaphoreType.DMA((2 * LOG2_N,)),
            pltpu.SemaphoreType.DMA((2 * LOG2_N,)),
        ],
        compiler_params=pltpu.CompilerParams(collective_id=0),
    )(x, Wq, Wo, K_ext, V_ext)


# baseline (device time: 123224 ns/iter reference)
import jax
import jax.numpy as jnp
from jax import lax
from jax.experimental import pallas as pl
from jax.experimental.pallas import tpu as pltpu

try:
    jax.config.update("jax_compilation_cache_dir", "/tmp/jaxcache")
    jax.config.update("jax_persistent_cache_min_compile_time_secs", 0)
    jax.config.update("jax_persistent_cache_min_entry_size_bytes", 0)
except Exception:
    pass
try:
    for _a in jax.live_arrays():
        jax.block_until_ready(_a)
except Exception:
    pass

N_DEV = 32
H_LOC = 8
DH = 128
SQ = 512
SKV = 2048
D_LOC = H_LOC * DH
SCALE = 0.08838834764831843
LOG2_N = 5

_RS_SLOT = [0, 256, 384, 448, 480]
_RS_ROWS = 496


def kernel(x, Wq, Wo, K_ext, V_ext):
    def body(x_ref, wq_ref, wo_ref, k_hbm, v_hbm, out_ref,
             k_vmem, v_vmem, attn_ref, recv_ref,
             copy_sems, send_sems, recv_sems):
        my = lax.axis_index("i")
        h0 = my * H_LOC

        kcp = pltpu.make_async_copy(
            k_hbm.at[0, :, pl.ds(h0, H_LOC), :], k_vmem, copy_sems.at[0])
        vcp = pltpu.make_async_copy(
            v_hbm.at[0, :, pl.ds(h0, H_LOC), :], v_vmem, copy_sems.at[1])
        kcp.start()
        vcp.start()

        q = jnp.dot(x_ref[0].astype(jnp.bfloat16),
                    wq_ref[:].astype(jnp.bfloat16),
                    preferred_element_type=jnp.float32)

        kcp.wait()
        vcp.wait()

        for h in range(H_LOC):
            qh = q[:, h * DH:(h + 1) * DH].astype(jnp.bfloat16)
            kh = k_vmem[:, h, :].astype(jnp.bfloat16)
            s = lax.dot_general(
                qh, kh, (((1,), (1,)), ((), ())),
                preferred_element_type=jnp.float32) * SCALE
            m = jnp.max(s, axis=1, keepdims=True)
            p = jnp.exp(s - m)
            l = jnp.sum(p, axis=1, keepdims=True)
            vh = v_vmem[:, h, :].astype(jnp.bfloat16)
            o = jnp.dot(p.astype(jnp.bfloat16), vh,
                        preferred_element_type=jnp.float32)
            attn_ref[:, h * DH:(h + 1) * DH] = (o / l).astype(jnp.bfloat16)

        out_ref[0] = jnp.dot(attn_ref[:], wo_ref[:].astype(jnp.bfloat16),
                             preferred_element_type=jnp.float32)

        off = 0
        ln = SQ
        for k in range(LOG2_N):
            half = ln // 2
            b = jnp.bitwise_and(jnp.right_shift(my, k), 1)
            partner = jnp.bitwise_xor(my, 1 << k)
            send_off = off + (1 - b) * half
            keep_off = off + b * half
            rdma = pltpu.make_async_remote_copy(
                src_ref=out_ref.at[0, pl.ds(send_off, half), :],
                dst_ref=recv_ref.at[pl.ds(_RS_SLOT[k], half), :],
                send_sem=send_sems.at[k],
                recv_sem=recv_sems.at[k],
                device_id=(partner,),
                device_id_type=pl.DeviceIdType.MESH,
            )
            rdma.start()
            rdma.wait()
            out_ref[0, pl.ds(keep_off, half), :] = (
                out_ref[0, pl.ds(keep_off, half), :]
                + recv_ref[_RS_SLOT[k]:_RS_SLOT[k] + half, :])
            off = keep_off
            ln = half

        for i, k in enumerate(range(LOG2_N - 1, -1, -1)):
            b = jnp.bitwise_and(jnp.right_shift(my, k), 1)
            partner = jnp.bitwise_xor(my, 1 << k)
            rdma = pltpu.make_async_remote_copy(
                src_ref=out_ref.at[0, pl.ds(off, ln), :],
                dst_ref=out_ref.at[0, pl.ds(off, ln), :],
                send_sem=send_sems.at[LOG2_N + i],
                recv_sem=recv_sems.at[LOG2_N + i],
                device_id=(partner,),
                device_id_type=pl.DeviceIdType.MESH,
            )
            rdma.start()
            rdma.wait()
            off = off - b * ln
            ln = ln * 2

    return pl.pallas_call(
        body,
        out_shape=jax.ShapeDtypeStruct((1, SQ, D_LOC), jnp.float32),
        in_specs=[
            pl.BlockSpec(memory_space=pltpu.VMEM),
            pl.BlockSpec(memory_space=pltpu.VMEM),
            pl.BlockSpec(memory_space=pltpu.VMEM),
            pl.BlockSpec(memory_space=pl.ANY),
            pl.BlockSpec(memory_space=pl.ANY),
        ],
        out_specs=pl.BlockSpec(memory_space=pltpu.VMEM),
        scratch_shapes=[
            pltpu.VMEM((SKV, H_LOC, DH), jnp.float32),
            pltpu.VMEM((SKV, H_LOC, DH), jnp.float32),
            pltpu.VMEM((SQ, D_LOC), jnp.bfloat16),
            pltpu.VMEM((_RS_ROWS, D_LOC), jnp.float32),
            pltpu.SemaphoreType.DMA((2,)),
            pltpu.SemaphoreType.DMA((2 * LOG2_N,)),
            pltpu.SemaphoreType.DMA((2 * LOG2_N,)),
        ],
        compiler_params=pltpu.CompilerParams(
            vmem_limit_bytes=100 * 1024 * 1024,
        ),
    )(x, Wq, Wo, K_ext, V_ext)


# device time: 98522 ns/iter; 1.2507x vs baseline; 1.2507x over previous
import jax
import jax.numpy as jnp
from jax import lax
from jax.experimental import pallas as pl
from jax.experimental.pallas import tpu as pltpu

try:
    jax.config.update("jax_compilation_cache_dir", "/tmp/jaxcache")
    jax.config.update("jax_persistent_cache_min_compile_time_secs", 0)
    jax.config.update("jax_persistent_cache_min_entry_size_bytes", 0)
except Exception:
    pass
try:
    for _a in jax.live_arrays():
        jax.block_until_ready(_a)
except Exception:
    pass

N_DEV = 32
H_LOC = 8
DH = 128
SQ = 512
SKV = 2048
D_LOC = H_LOC * DH
SCALE = 0.08838834764831843
LOG2_N = 5

_RS_SLOT = [0, 256, 384, 448, 480]
_RS_ROWS = 496


def kernel(x, Wq, Wo, K_ext, V_ext):
    def body(x_ref, wq_ref, wo_ref, k_hbm, v_hbm, out_ref,
             k_vmem, v_vmem, attn_ref, acc_ref, recv_ref,
             copy_sems, send_sems, recv_sems):
        my = lax.axis_index("i")
        h0 = my * H_LOC

        kcp = pltpu.make_async_copy(
            k_hbm.at[0, :, pl.ds(h0, H_LOC), :], k_vmem, copy_sems.at[0])
        vcp = pltpu.make_async_copy(
            v_hbm.at[0, :, pl.ds(h0, H_LOC), :], v_vmem, copy_sems.at[1])
        kcp.start()
        vcp.start()

        q = jnp.dot(x_ref[0].astype(jnp.bfloat16),
                    wq_ref[:].astype(jnp.bfloat16),
                    preferred_element_type=jnp.float32)

        kcp.wait()
        vcp.wait()

        for h in range(H_LOC):
            qh = q[:, h * DH:(h + 1) * DH].astype(jnp.bfloat16)
            kh = k_vmem[:, h, :].astype(jnp.bfloat16)
            s = lax.dot_general(
                qh, kh, (((1,), (1,)), ((), ())),
                preferred_element_type=jnp.float32) * SCALE
            m = jnp.max(s, axis=1, keepdims=True)
            p = jnp.exp(s - m)
            l = jnp.sum(p, axis=1, keepdims=True)
            vh = v_vmem[:, h, :].astype(jnp.bfloat16)
            o = jnp.dot(p.astype(jnp.bfloat16), vh,
                        preferred_element_type=jnp.float32)
            attn_ref[:, h * DH:(h + 1) * DH] = (o / l).astype(jnp.bfloat16)

        acc_ref[:] = jnp.dot(attn_ref[:], wo_ref[:].astype(jnp.bfloat16),
                             preferred_element_type=jnp.float32
                             ).astype(jnp.bfloat16)

        off = 0
        ln = SQ
        for k in range(LOG2_N):
            half = ln // 2
            b = jnp.bitwise_and(jnp.right_shift(my, k), 1)
            partner = jnp.bitwise_xor(my, 1 << k)
            send_off = off + (1 - b) * half
            keep_off = off + b * half
            rdma = pltpu.make_async_remote_copy(
                src_ref=acc_ref.at[pl.ds(send_off, half), :],
                dst_ref=recv_ref.at[pl.ds(_RS_SLOT[k], half), :],
                send_sem=send_sems.at[k],
                recv_sem=recv_sems.at[k],
                device_id=(partner,),
                device_id_type=pl.DeviceIdType.MESH,
            )
            rdma.start()
            rdma.wait()
            acc_ref[pl.ds(keep_off, half), :] = (
                acc_ref[pl.ds(keep_off, half), :]
                + recv_ref[_RS_SLOT[k]:_RS_SLOT[k] + half, :])
            off = keep_off
            ln = half

        for i, k in enumerate(range(LOG2_N - 1, -1, -1)):
            b = jnp.bitwise_and(jnp.right_shift(my, k), 1)
            partner = jnp.bitwise_xor(my, 1 << k)
            rdma = pltpu.make_async_remote_copy(
                src_ref=acc_ref.at[pl.ds(off, ln), :],
                dst_ref=acc_ref.at[pl.ds(off, ln), :],
                send_sem=send_sems.at[LOG2_N + i],
                recv_sem=recv_sems.at[LOG2_N + i],
                device_id=(partner,),
                device_id_type=pl.DeviceIdType.MESH,
            )
            rdma.start()
            rdma.wait()
            off = off - b * ln
            ln = ln * 2

        out_ref[0] = acc_ref[:].astype(jnp.float32)

    return pl.pallas_call(
        body,
        out_shape=jax.ShapeDtypeStruct((1, SQ, D_LOC), jnp.float32),
        in_specs=[
            pl.BlockSpec(memory_space=pltpu.VMEM),
            pl.BlockSpec(memory_space=pltpu.VMEM),
            pl.BlockSpec(memory_space=pltpu.VMEM),
            pl.BlockSpec(memory_space=pl.ANY),
            pl.BlockSpec(memory_space=pl.ANY),
        ],
        out_specs=pl.BlockSpec(memory_space=pltpu.VMEM),
        scratch_shapes=[
            pltpu.VMEM((SKV, H_LOC, DH), jnp.float32),
            pltpu.VMEM((SKV, H_LOC, DH), jnp.float32),
            pltpu.VMEM((SQ, D_LOC), jnp.bfloat16),
            pltpu.VMEM((SQ, D_LOC), jnp.bfloat16),
            pltpu.VMEM((_RS_ROWS, D_LOC), jnp.bfloat16),
            pltpu.SemaphoreType.DMA((2,)),
            pltpu.SemaphoreType.DMA((2 * LOG2_N,)),
            pltpu.SemaphoreType.DMA((2 * LOG2_N,)),
        ],
        compiler_params=pltpu.CompilerParams(
            vmem_limit_bytes=100 * 1024 * 1024,
        ),
    )(x, Wq, Wo, K_ext, V_ext)
